# baseline (speedup 1.0000x reference)
import jax
import jax.numpy as jnp
from jax import lax
from jax.experimental import pallas as pl
from jax.experimental.pallas import tpu as pltpu
from jax.experimental.pallas import tpu_sc as plsc

B, N, K, D = 16, 4096, 20, 64
NC, NS, L = 2, 16, 16
NW = NC * NS
DT = D // 8
PAIRS_PER_W = (B * DT) // NW
NT = N // 128


def _sc_gather(idx_hbm, feat_hbm, out_hbm, slab, idx_v, buf):
    wid = lax.axis_index("s") * NC + lax.axis_index("c")

    for p in range(PAIRS_PER_W):
        pair = wid * PAIRS_PER_W + p
        b = pair // DT
        dt = pair % DT
        pltpu.sync_copy(feat_hbm.at[b, :, pl.ds(dt * 8, 8)], slab)

        def k_body(k, _):
            pltpu.sync_copy(idx_hbm.at[b, k], idx_v)

            def g_body(g, _):
                iv = idx_v[pl.ds(g * L, L)]
                nt = g // 8
                j = g % 8
                sl = pl.ds(j * L, L)
                for di in range(8):
                    col = jnp.full((L,), di, jnp.int32)
                    buf[nt, di, sl] = plsc.load_gather(slab, [iv, col])
                return _

            lax.fori_loop(0, NT * 8, g_body, 0)
            pltpu.sync_copy(buf, out_hbm.at[b, k, dt])
            return _

        lax.fori_loop(0, K, k_body, 0)


@jax.jit
def kernel(topk_indices, features):
    idx = topk_indices.astype(jnp.int32).transpose(0, 2, 1)
    mesh = plsc.VectorSubcoreMesh(core_axis_name="c", subcore_axis_name="s")
    out6 = pl.kernel(
        _sc_gather,
        out_type=jax.ShapeDtypeStruct((B, K, DT, NT, 8, 128), jnp.float32),
        mesh=mesh,
        scratch_types=[
            pltpu.VMEM((N, 8), jnp.float32),
            pltpu.VMEM((N,), jnp.int32),
            pltpu.VMEM((NT, 8, 128), jnp.float32),
        ],
        compiler_params=pltpu.CompilerParams(
            use_tc_tiling_on_sc=False, needs_layout_passes=False
        ),
    )(idx, features)
    return out6.transpose(0, 3, 5, 1, 2, 4).reshape(B, N, K, D)
